# baseline (speedup 1.0000x reference)
import functools

import jax
import jax.numpy as jnp
from jax import lax
from jax.experimental import pallas as pl
from jax.experimental.pallas import tpu as pltpu
from jax.experimental.pallas import tpu_sc as plsc

D = 1024
DFF = 4096
E = 64
DEXP = 1024
FF_BLK = 1024
N_FF = DFF // FF_BLK
NEG_BIG = -3.0e38
DCHUNK = DEXP // 2
T = 128

N_CORES = 2
LANES = 16
N_ACTIVE = T // LANES
G = 8


def _ffn_body(x_ref, Wfc1_ref, bfc1_ref, Wfc2_ref, bfc2_ref, Wg_ref,
              bg_ref, tokens_ref, logitsT_ref, acc_ref):
    i = pl.program_id(0)
    h = jnp.dot(x_ref[...], Wfc1_ref[...], preferred_element_type=jnp.float32)
    h = jnp.maximum(h + bfc1_ref[0], 0.0)
    contrib = jnp.dot(h, Wfc2_ref[...], preferred_element_type=jnp.float32)

    @pl.when(i == 0)
    def _():
        acc_ref[...] = contrib

    @pl.when(i > 0)
    def _():
        acc_ref[...] += contrib

    @pl.when(i == pl.num_programs(0) - 1)
    def _():
        tokens = acc_ref[...] + bfc2_ref[...]
        tokens_ref[...] = tokens
        logitsT = jax.lax.dot_general(
            Wg_ref[...], tokens, (((0,), (1,)), ((), ())),
            preferred_element_type=jnp.float32)
        logitsT = logitsT + bg_ref[...]
        for b in range(N_ACTIVE):
            logitsT_ref[b] = logitsT[:, b * LANES:(b + 1) * LANES]


def _router_sc_body(logitsT_hbm, wT_hbm, lg_v, w_v):
    wid = lax.axis_index("s") * N_CORES + lax.axis_index("c")

    @pl.when(wid < N_ACTIVE)
    def _():
        pltpu.sync_copy(logitsT_hbm.at[wid], lg_v)
        m1 = lg_v[0, :]
        i1 = jnp.zeros((LANES,), jnp.int32)
        for e in range(1, E):
            v = lg_v[e, :]
            better = v > m1
            m1 = jnp.where(better, v, m1)
            i1 = jnp.where(better, e, i1)
        m2 = jnp.full((LANES,), NEG_BIG, jnp.float32)
        i2 = jnp.zeros((LANES,), jnp.int32)
        for e in range(E):
            v = jnp.where(i1 == e, NEG_BIG, lg_v[e, :])
            better = v > m2
            m2 = jnp.where(better, v, m2)
            i2 = jnp.where(better, e, i2)
        ev = jnp.exp(m2 - m1)
        denom = 1.0 + ev
        g1 = 1.0 / denom
        g2 = ev / denom
        zero = jnp.zeros((LANES,), jnp.float32)
        for e in range(E):
            w_v[e, :] = jnp.where(i1 == e, g1, zero) + \
                        jnp.where(i2 == e, g2, zero)
        pltpu.sync_copy(w_v, wT_hbm.at[wid])


def _make_router_sc():
    return pl.kernel(
        _router_sc_body,
        mesh=plsc.VectorSubcoreMesh(core_axis_name="c", subcore_axis_name="s"),
        out_type=jax.ShapeDtypeStruct((N_ACTIVE, E, LANES), jnp.float32),
        scratch_types=[
            pltpu.VMEM((E, LANES), jnp.float32),
            pltpu.VMEM((E, LANES), jnp.float32),
        ],
    )


def _pre_expert_body(tokens_ref, W1a_ref, W1b_ref, b1a_ref, b1b_ref,
                     W2a_ref, W2b_ref, outA_ref):
    acc = None
    for W1h_ref, b1h_ref, W2h_ref in ((W1a_ref, b1a_ref, W2a_ref),
                                      (W1b_ref, b1b_ref, W2b_ref)):
        h1 = jnp.dot(tokens_ref[...], W1h_ref[0],
                     preferred_element_type=jnp.float32)
        h1 = jnp.maximum(h1 + b1h_ref[0], 0.0)
        c = jnp.dot(h1, W2h_ref[0], preferred_element_type=jnp.float32)
        acc = c if acc is None else acc + c
    outA_ref[0] = acc


def _wcol(wT, j):
    onehot = (jax.lax.broadcasted_iota(jnp.int32, (E, 1), 0) == j
              ).astype(jnp.float32)
    return jax.lax.dot_general(wT, onehot, (((0,), (0,)), ((), ())),
                               preferred_element_type=jnp.float32)


def _expert_body(tokens_ref, w3_ref, outA_ref, W1a_ref, W1b_ref, b1a_ref,
                 b1b_ref, W2a_ref, W2b_ref, b2_ref, out_ref, acc_ref,
                 wT_ref):
    k = pl.program_id(0)
    e = k + G

    @pl.when(k == 0)
    def _():
        w3 = w3_ref[...]
        wT = jnp.concatenate([w3[b] for b in range(N_ACTIVE)], axis=1)
        wT_ref[...] = wT
        acc_ref[...] = jax.lax.dot_general(
            wT, b2_ref[...], (((0,), (0,)), ((), ())),
            preferred_element_type=jnp.float32)

    @pl.when(k < G)
    def _():
        acc_ref[...] += _wcol(wT_ref[...], k) * outA_ref[0]

    contrib = acc_ref[...]
    wcol = _wcol(wT_ref[...], e)
    for W1h_ref, b1h_ref, W2h_ref in ((W1a_ref, b1a_ref, W2a_ref),
                                      (W1b_ref, b1b_ref, W2b_ref)):
        h1 = jnp.dot(tokens_ref[...], W1h_ref[0],
                     preferred_element_type=jnp.float32)
        h1 = jnp.maximum(h1 + b1h_ref[0], 0.0)
        contrib += jnp.dot(h1 * wcol, W2h_ref[0],
                           preferred_element_type=jnp.float32)
    acc_ref[...] = contrib

    @pl.when(k == pl.num_programs(0) - 1)
    def _():
        out_ref[...] = acc_ref[...]


def kernel(x, W_fc1, b_fc1, W_fc2, b_fc2, Wg, bg, W1, b1, W2, b2):
    B, S, _ = x.shape
    xt = x.reshape(T, D)

    tokens, logitsT = pl.pallas_call(
        _ffn_body,
        grid=(N_FF,),
        in_specs=[
            pl.BlockSpec((T, D), lambda f: (0, 0)),
            pl.BlockSpec((D, FF_BLK), lambda f: (0, f)),
            pl.BlockSpec((1, 1, FF_BLK), lambda f: (f, 0, 0)),
            pl.BlockSpec((FF_BLK, D), lambda f: (f, 0)),
            pl.BlockSpec((1, D), lambda f: (0, 0)),
            pl.BlockSpec((D, E), lambda f: (0, 0)),
            pl.BlockSpec((E, 1), lambda f: (0, 0)),
        ],
        out_specs=[
            pl.BlockSpec((T, D), lambda f: (0, 0)),
            pl.BlockSpec((N_ACTIVE, E, LANES), lambda f: (0, 0, 0)),
        ],
        out_shape=[
            jax.ShapeDtypeStruct((T, D), jnp.float32),
            jax.ShapeDtypeStruct((N_ACTIVE, E, LANES), jnp.float32),
        ],
        scratch_shapes=[pltpu.VMEM((T, D), jnp.float32)],
    )(xt, W_fc1, b_fc1.reshape(N_FF, 1, FF_BLK), W_fc2,
      b_fc2.reshape(1, D), Wg, bg.reshape(E, 1))

    w3 = _make_router_sc()(logitsT)

    b1r = b1.reshape(E, 1, DEXP)
    outA = pl.pallas_call(
        _pre_expert_body,
        grid=(G,),
        in_specs=[
            pl.BlockSpec((T, D), lambda e: (0, 0)),
            pl.BlockSpec((1, D, DCHUNK), lambda e: (e, 0, 0)),
            pl.BlockSpec((1, D, DCHUNK), lambda e: (e, 0, 1)),
            pl.BlockSpec((1, 1, DCHUNK), lambda e: (e, 0, 0)),
            pl.BlockSpec((1, 1, DCHUNK), lambda e: (e, 0, 1)),
            pl.BlockSpec((1, DCHUNK, D), lambda e: (e, 0, 0)),
            pl.BlockSpec((1, DCHUNK, D), lambda e: (e, 1, 0)),
        ],
        out_specs=pl.BlockSpec((1, T, D), lambda e: (e, 0, 0)),
        out_shape=jax.ShapeDtypeStruct((G, T, D), jnp.float32),
    )(tokens, W1, W1, b1r, b1r, W2, W2)

    out = pl.pallas_call(
        _expert_body,
        grid=(E - G,),
        in_specs=[
            pl.BlockSpec((T, D), lambda k: (0, 0)),
            pl.BlockSpec((N_ACTIVE, E, LANES), lambda k: (0, 0, 0)),
            pl.BlockSpec((1, T, D), lambda k: (jnp.minimum(k, G - 1), 0, 0)),
            pl.BlockSpec((1, D, DCHUNK), lambda k: (k + G, 0, 0)),
            pl.BlockSpec((1, D, DCHUNK), lambda k: (k + G, 0, 1)),
            pl.BlockSpec((1, 1, DCHUNK), lambda k: (k + G, 0, 0)),
            pl.BlockSpec((1, 1, DCHUNK), lambda k: (k + G, 0, 1)),
            pl.BlockSpec((1, DCHUNK, D), lambda k: (k + G, 0, 0)),
            pl.BlockSpec((1, DCHUNK, D), lambda k: (k + G, 1, 0)),
            pl.BlockSpec((E, D), lambda k: (0, 0)),
        ],
        out_specs=pl.BlockSpec((T, D), lambda k: (0, 0)),
        out_shape=jax.ShapeDtypeStruct((T, D), jnp.float32),
        scratch_shapes=[
            pltpu.VMEM((T, D), jnp.float32),
            pltpu.VMEM((E, T), jnp.float32),
        ],
    )(tokens, w3, outA, W1, W1, b1r, b1r, W2, W2, b2)

    return (out.reshape(B, S, D),)


# speedup vs baseline: 1.0208x; 1.0208x over previous
import jax
import jax.numpy as jnp
from jax import lax
from jax.experimental import pallas as pl
from jax.experimental.pallas import tpu as pltpu
from jax.experimental.pallas import tpu_sc as plsc

D = 1024
DFF = 4096
E = 64
DEXP = 1024
FF_BLK = 1024
N_FF = DFF // FF_BLK
NEG_BIG = -3.0e38
DCHUNK = DEXP // 2
T = 128

N_CORES = 2
LANES = 16
N_ACTIVE = T // LANES


def _ffn_body(x_ref, Wfc1_ref, bfc1_ref, Wfc2_ref, bfc2_ref, Wg_ref,
              bg_ref, tokens_ref, logitsT_ref, acc_ref):
    i = pl.program_id(0)
    h = jnp.dot(x_ref[...], Wfc1_ref[...], preferred_element_type=jnp.float32)
    h = jnp.maximum(h + bfc1_ref[0], 0.0)
    contrib = jnp.dot(h, Wfc2_ref[...], preferred_element_type=jnp.float32)

    @pl.when(i == 0)
    def _():
        acc_ref[...] = contrib

    @pl.when(i > 0)
    def _():
        acc_ref[...] += contrib

    @pl.when(i == pl.num_programs(0) - 1)
    def _():
        tokens = acc_ref[...] + bfc2_ref[...]
        tokens_ref[...] = tokens
        logitsT = jax.lax.dot_general(
            Wg_ref[...], tokens, (((0,), (1,)), ((), ())),
            preferred_element_type=jnp.float32)
        logitsT = logitsT + bg_ref[...]
        for b in range(N_ACTIVE):
            logitsT_ref[b] = logitsT[:, b * LANES:(b + 1) * LANES]


def _router_sc_body(logitsT_hbm, wT_hbm, lg_v, w_v):
    wid = lax.axis_index("s") * N_CORES + lax.axis_index("c")

    @pl.when(wid < N_ACTIVE)
    def _():
        pltpu.sync_copy(logitsT_hbm.at[wid], lg_v)
        m1 = lg_v[0, :]
        i1 = jnp.zeros((LANES,), jnp.int32)
        for e in range(1, E):
            v = lg_v[e, :]
            better = v > m1
            m1 = jnp.where(better, v, m1)
            i1 = jnp.where(better, e, i1)
        m2 = jnp.full((LANES,), NEG_BIG, jnp.float32)
        i2 = jnp.zeros((LANES,), jnp.int32)
        for e in range(E):
            v = jnp.where(i1 == e, NEG_BIG, lg_v[e, :])
            better = v > m2
            m2 = jnp.where(better, v, m2)
            i2 = jnp.where(better, e, i2)
        ev = jnp.exp(m2 - m1)
        denom = 1.0 + ev
        g1 = 1.0 / denom
        g2 = ev / denom
        zero = jnp.zeros((LANES,), jnp.float32)
        for e in range(E):
            w_v[e, :] = jnp.where(i1 == e, g1, zero) + \
                        jnp.where(i2 == e, g2, zero)
        pltpu.sync_copy(w_v, wT_hbm.at[wid])


def _make_router_sc():
    return pl.kernel(
        _router_sc_body,
        mesh=plsc.VectorSubcoreMesh(core_axis_name="c", subcore_axis_name="s"),
        out_type=jax.ShapeDtypeStruct((N_ACTIVE, E, LANES), jnp.float32),
        scratch_types=[
            pltpu.VMEM((E, LANES), jnp.float32),
            pltpu.VMEM((E, LANES), jnp.float32),
        ],
    )


def _expert_body(tokens_ref, w3_ref, W1a_ref, W1b_ref, b1a_ref, b1b_ref,
                 W2a_ref, W2b_ref, b2_ref, out_ref, acc_ref, wT_ref):
    e = pl.program_id(0)

    @pl.when(e == 0)
    def _():
        w3 = w3_ref[...]
        wT = jnp.concatenate([w3[b] for b in range(N_ACTIVE)], axis=1)
        wT_ref[...] = wT
        acc_ref[...] = jax.lax.dot_general(
            wT, b2_ref[...], (((0,), (0,)), ((), ())),
            preferred_element_type=jnp.float32)

    onehot = (jax.lax.broadcasted_iota(jnp.int32, (E, 1), 0) == e
              ).astype(jnp.float32)
    wcol = jax.lax.dot_general(wT_ref[...], onehot, (((0,), (0,)), ((), ())),
                               preferred_element_type=jnp.float32)
    contrib = acc_ref[...]
    for W1h_ref, b1h_ref, W2h_ref in ((W1a_ref, b1a_ref, W2a_ref),
                                      (W1b_ref, b1b_ref, W2b_ref)):
        h1 = jnp.dot(tokens_ref[...], W1h_ref[0],
                     preferred_element_type=jnp.float32)
        h1 = jnp.maximum(h1 + b1h_ref[0], 0.0)
        contrib += jnp.dot(h1 * wcol, W2h_ref[0],
                           preferred_element_type=jnp.float32)
    acc_ref[...] = contrib

    @pl.when(e == pl.num_programs(0) - 1)
    def _():
        out_ref[...] = acc_ref[...]


def kernel(x, W_fc1, b_fc1, W_fc2, b_fc2, Wg, bg, W1, b1, W2, b2):
    B, S, _ = x.shape
    xt = x.reshape(T, D)

    tokens, logitsT = pl.pallas_call(
        _ffn_body,
        grid=(N_FF,),
        in_specs=[
            pl.BlockSpec((T, D), lambda f: (0, 0)),
            pl.BlockSpec((D, FF_BLK), lambda f: (0, f)),
            pl.BlockSpec((1, 1, FF_BLK), lambda f: (f, 0, 0)),
            pl.BlockSpec((FF_BLK, D), lambda f: (f, 0)),
            pl.BlockSpec((1, D), lambda f: (0, 0)),
            pl.BlockSpec((D, E), lambda f: (0, 0)),
            pl.BlockSpec((E, 1), lambda f: (0, 0)),
        ],
        out_specs=[
            pl.BlockSpec((T, D), lambda f: (0, 0)),
            pl.BlockSpec((N_ACTIVE, E, LANES), lambda f: (0, 0, 0)),
        ],
        out_shape=[
            jax.ShapeDtypeStruct((T, D), jnp.float32),
            jax.ShapeDtypeStruct((N_ACTIVE, E, LANES), jnp.float32),
        ],
        scratch_shapes=[pltpu.VMEM((T, D), jnp.float32)],
    )(xt, W_fc1, b_fc1.reshape(N_FF, 1, FF_BLK), W_fc2,
      b_fc2.reshape(1, D), Wg, bg.reshape(E, 1))

    w3 = _make_router_sc()(logitsT)

    out = pl.pallas_call(
        _expert_body,
        grid=(E,),
        in_specs=[
            pl.BlockSpec((T, D), lambda e: (0, 0)),
            pl.BlockSpec((N_ACTIVE, E, LANES), lambda e: (0, 0, 0)),
            pl.BlockSpec((1, D, DCHUNK), lambda e: (e, 0, 0)),
            pl.BlockSpec((1, D, DCHUNK), lambda e: (e, 0, 1)),
            pl.BlockSpec((1, 1, DCHUNK), lambda e: (e, 0, 0)),
            pl.BlockSpec((1, 1, DCHUNK), lambda e: (e, 0, 1)),
            pl.BlockSpec((1, DCHUNK, D), lambda e: (e, 0, 0)),
            pl.BlockSpec((1, DCHUNK, D), lambda e: (e, 1, 0)),
            pl.BlockSpec((E, D), lambda e: (0, 0)),
        ],
        out_specs=pl.BlockSpec((T, D), lambda e: (0, 0)),
        out_shape=jax.ShapeDtypeStruct((T, D), jnp.float32),
        scratch_shapes=[
            pltpu.VMEM((T, D), jnp.float32),
            pltpu.VMEM((E, T), jnp.float32),
        ],
    )(tokens, w3, W1, W1, b1.reshape(E, 1, DEXP), b1.reshape(E, 1, DEXP),
      W2, W2, b2)

    return (out.reshape(B, S, D),)


# speedup vs baseline: 1.0261x; 1.0053x over previous
import jax
import jax.numpy as jnp
from jax import lax
from jax.experimental import pallas as pl
from jax.experimental.pallas import tpu as pltpu
from jax.experimental.pallas import tpu_sc as plsc

D = 1024
DFF = 4096
E = 64
DEXP = 1024
FF_BLK = 2048
N_FF = DFF // FF_BLK
NEG_BIG = -3.0e38
DCHUNK = DEXP // 2
T = 128

N_CORES = 2
LANES = 16
N_ACTIVE = T // LANES


def _ffn_body(x_ref, Wfc1_ref, bfc1_ref, Wfc2_ref, bfc2_ref, Wg_ref,
              bg_ref, tokens_ref, logitsT_ref, acc_ref):
    i = pl.program_id(0)
    h = jnp.dot(x_ref[...], Wfc1_ref[...], preferred_element_type=jnp.float32)
    h = jnp.maximum(h + bfc1_ref[0], 0.0)
    contrib = jnp.dot(h, Wfc2_ref[...], preferred_element_type=jnp.float32)

    @pl.when(i == 0)
    def _():
        acc_ref[...] = contrib

    @pl.when(i > 0)
    def _():
        acc_ref[...] += contrib

    @pl.when(i == pl.num_programs(0) - 1)
    def _():
        tokens = acc_ref[...] + bfc2_ref[...]
        tokens_ref[...] = tokens
        logitsT = jax.lax.dot_general(
            Wg_ref[...], tokens, (((0,), (1,)), ((), ())),
            preferred_element_type=jnp.float32)
        logitsT = logitsT + bg_ref[...]
        for b in range(N_ACTIVE):
            logitsT_ref[b] = logitsT[:, b * LANES:(b + 1) * LANES]


def _router_sc_body(logitsT_hbm, wT_hbm, lg_v, w_v):
    wid = lax.axis_index("s") * N_CORES + lax.axis_index("c")

    @pl.when(wid < N_ACTIVE)
    def _():
        pltpu.sync_copy(logitsT_hbm.at[wid], lg_v)
        m1 = lg_v[0, :]
        i1 = jnp.zeros((LANES,), jnp.int32)
        for e in range(1, E):
            v = lg_v[e, :]
            better = v > m1
            m1 = jnp.where(better, v, m1)
            i1 = jnp.where(better, e, i1)
        m2 = jnp.full((LANES,), NEG_BIG, jnp.float32)
        i2 = jnp.zeros((LANES,), jnp.int32)
        for e in range(E):
            v = jnp.where(i1 == e, NEG_BIG, lg_v[e, :])
            better = v > m2
            m2 = jnp.where(better, v, m2)
            i2 = jnp.where(better, e, i2)
        ev = jnp.exp(m2 - m1)
        denom = 1.0 + ev
        g1 = 1.0 / denom
        g2 = ev / denom
        zero = jnp.zeros((LANES,), jnp.float32)
        for e in range(E):
            w_v[e, :] = jnp.where(i1 == e, g1, zero) + \
                        jnp.where(i2 == e, g2, zero)
        pltpu.sync_copy(w_v, wT_hbm.at[wid])


def _make_router_sc():
    return pl.kernel(
        _router_sc_body,
        mesh=plsc.VectorSubcoreMesh(core_axis_name="c", subcore_axis_name="s"),
        out_type=jax.ShapeDtypeStruct((N_ACTIVE, E, LANES), jnp.float32),
        scratch_types=[
            pltpu.VMEM((E, LANES), jnp.float32),
            pltpu.VMEM((E, LANES), jnp.float32),
        ],
    )


def _expert_body(tokens_ref, w3_ref, W1a_ref, W1b_ref, b1a_ref, b1b_ref,
                 W2a_ref, W2b_ref, b2_ref, out_ref, acc_ref, wT_ref):
    e = pl.program_id(0)

    @pl.when(e == 0)
    def _():
        w3 = w3_ref[...]
        wT = jnp.concatenate([w3[b] for b in range(N_ACTIVE)], axis=1)
        wT_ref[...] = wT
        acc_ref[...] = jax.lax.dot_general(
            wT, b2_ref[...], (((0,), (0,)), ((), ())),
            preferred_element_type=jnp.float32)

    onehot = (jax.lax.broadcasted_iota(jnp.int32, (E, 1), 0) == e
              ).astype(jnp.float32)
    wcol = jax.lax.dot_general(wT_ref[...], onehot, (((0,), (0,)), ((), ())),
                               preferred_element_type=jnp.float32)
    contrib = acc_ref[...]
    for W1h_ref, b1h_ref, W2h_ref in ((W1a_ref, b1a_ref, W2a_ref),
                                      (W1b_ref, b1b_ref, W2b_ref)):
        h1 = jnp.dot(tokens_ref[...], W1h_ref[0],
                     preferred_element_type=jnp.float32)
        h1 = jnp.maximum(h1 + b1h_ref[0], 0.0)
        contrib += jnp.dot(h1 * wcol, W2h_ref[0],
                           preferred_element_type=jnp.float32)
    acc_ref[...] = contrib

    @pl.when(e == pl.num_programs(0) - 1)
    def _():
        out_ref[...] = acc_ref[...]


def kernel(x, W_fc1, b_fc1, W_fc2, b_fc2, Wg, bg, W1, b1, W2, b2):
    B, S, _ = x.shape
    xt = x.reshape(T, D)

    tokens, logitsT = pl.pallas_call(
        _ffn_body,
        grid=(N_FF,),
        in_specs=[
            pl.BlockSpec((T, D), lambda f: (0, 0)),
            pl.BlockSpec((D, FF_BLK), lambda f: (0, f)),
            pl.BlockSpec((1, 1, FF_BLK), lambda f: (f, 0, 0)),
            pl.BlockSpec((FF_BLK, D), lambda f: (f, 0)),
            pl.BlockSpec((1, D), lambda f: (0, 0)),
            pl.BlockSpec((D, E), lambda f: (0, 0)),
            pl.BlockSpec((E, 1), lambda f: (0, 0)),
        ],
        out_specs=[
            pl.BlockSpec((T, D), lambda f: (0, 0)),
            pl.BlockSpec((N_ACTIVE, E, LANES), lambda f: (0, 0, 0)),
        ],
        out_shape=[
            jax.ShapeDtypeStruct((T, D), jnp.float32),
            jax.ShapeDtypeStruct((N_ACTIVE, E, LANES), jnp.float32),
        ],
        scratch_shapes=[pltpu.VMEM((T, D), jnp.float32)],
    )(xt, W_fc1, b_fc1.reshape(N_FF, 1, FF_BLK), W_fc2,
      b_fc2.reshape(1, D), Wg, bg.reshape(E, 1))

    w3 = _make_router_sc()(logitsT)

    out = pl.pallas_call(
        _expert_body,
        grid=(E,),
        in_specs=[
            pl.BlockSpec((T, D), lambda e: (0, 0)),
            pl.BlockSpec((N_ACTIVE, E, LANES), lambda e: (0, 0, 0)),
            pl.BlockSpec((1, D, DCHUNK), lambda e: (e, 0, 0)),
            pl.BlockSpec((1, D, DCHUNK), lambda e: (e, 0, 1)),
            pl.BlockSpec((1, 1, DCHUNK), lambda e: (e, 0, 0)),
            pl.BlockSpec((1, 1, DCHUNK), lambda e: (e, 0, 1)),
            pl.BlockSpec((1, DCHUNK, D), lambda e: (e, 0, 0)),
            pl.BlockSpec((1, DCHUNK, D), lambda e: (e, 1, 0)),
            pl.BlockSpec((E, D), lambda e: (0, 0)),
        ],
        out_specs=pl.BlockSpec((T, D), lambda e: (0, 0)),
        out_shape=jax.ShapeDtypeStruct((T, D), jnp.float32),
        scratch_shapes=[
            pltpu.VMEM((T, D), jnp.float32),
            pltpu.VMEM((E, T), jnp.float32),
        ],
    )(tokens, w3, W1, W1, b1.reshape(E, 1, DEXP), b1.reshape(E, 1, DEXP),
      W2, W2, b2)

    return (out.reshape(B, S, D),)
